# Initial kernel scaffold; baseline (speedup 1.0000x reference)
import jax
import jax.numpy as jnp
from jax.experimental import pallas as pl


def kernel(x, bucket_assignment, emb0, emb1, emb2, emb3, emb4, W3, b3, W4, b4):
    raise NotImplementedError("write your pallas kernel here")


# speedup vs baseline: 14.7130x; 14.7130x over previous
import functools

import jax
import jax.numpy as jnp
from jax import lax
from jax.experimental import pallas as pl
from jax.experimental.pallas import tpu as pltpu
from jax.experimental.pallas import tpu_sc as plsc

V = 100000
BASE = 32
NB = 5
DIMS = (32, 32, 32, 51, 102)


_R = 800
_GRID = V // _R


def _build_body(ba_ref, e0_ref, e1_ref, e2_ref, e3_ref, e4_ref,
                w3t_ref, b3_ref, w4t_ref, b4_ref, out_ref):
    ba = ba_ref[0, 0, :].reshape(_R, 1)
    p3 = jnp.dot(e3_ref[...], w3t_ref[...],
                 preferred_element_type=jnp.float32) + b3_ref[...]
    p4 = jnp.dot(e4_ref[...], w4t_ref[...],
                 preferred_element_type=jnp.float32) + b4_ref[...]
    out = jnp.where(ba == 0, e0_ref[...], 0.0)
    out = out + jnp.where(ba == 1, e1_ref[...], 0.0)
    out = out + jnp.where(ba == 2, e2_ref[...], 0.0)
    out = out + jnp.where(ba == 3, p3, 0.0)
    out = out + jnp.where(ba == 4, p4, 0.0)
    out_ref[...] = out


def _build_combined(ba, emb0, emb1, emb2, emb3, emb4, w3t, b3, w4t, b4):
    ba3 = ba.reshape(_GRID, 1, _R).astype(jnp.int32)
    b3r = b3.reshape(1, BASE)
    b4r = b4.reshape(1, BASE)
    row = lambda i: (i, 0)
    fixed = lambda i: (0, 0)
    return pl.pallas_call(
        _build_body,
        grid=(_GRID,),
        in_specs=[
            pl.BlockSpec((1, 1, _R), lambda i: (i, 0, 0)),
            pl.BlockSpec((_R, DIMS[0]), row),
            pl.BlockSpec((_R, DIMS[1]), row),
            pl.BlockSpec((_R, DIMS[2]), row),
            pl.BlockSpec((_R, DIMS[3]), row),
            pl.BlockSpec((_R, DIMS[4]), row),
            pl.BlockSpec((DIMS[3], BASE), fixed),
            pl.BlockSpec((1, BASE), fixed),
            pl.BlockSpec((DIMS[4], BASE), fixed),
            pl.BlockSpec((1, BASE), fixed),
        ],
        out_specs=pl.BlockSpec((_R, BASE), row),
        out_shape=jax.ShapeDtypeStruct((V, BASE), jnp.float32),
    )(ba3, emb0, emb1, emb2, emb3, emb4, w3t, b3r, w4t, b4r)


_NC = 2
_NS = 16
_NW = _NC * _NS
_NTOK = 16384 * 50
_PER_W = _NTOK // _NW
_CH = 2560
_NCHUNK = _PER_W // _CH


@functools.partial(
    pl.kernel,
    mesh=plsc.VectorSubcoreMesh(core_axis_name="c", subcore_axis_name="s",
                                num_cores=_NC),
    out_type=jax.ShapeDtypeStruct((_NTOK, BASE), jnp.float32),
    scratch_types=[
        pltpu.VMEM((_CH,), jnp.int32),
        pltpu.VMEM((_CH, BASE), jnp.float32),
        pltpu.SemaphoreType.DMA,
    ],
    compiler_params=pltpu.CompilerParams(use_tc_tiling_on_sc=False),
)
def _sc_gather(c_hbm, idx_hbm, out_hbm, idxc, rows, sem):
    wid = lax.axis_index("s") * _NC + lax.axis_index("c")
    base = wid * _PER_W
    for c in range(_NCHUNK):
        off = base + c * _CH
        pltpu.sync_copy(idx_hbm.at[pl.ds(off, _CH)], idxc)
        pltpu.async_copy(c_hbm.at[idxc], rows, sem).wait()
        pltpu.sync_copy(rows, out_hbm.at[pl.ds(off, _CH)])


def kernel(x, bucket_assignment, emb0, emb1, emb2, emb3, emb4, W3, b3, W4, b4):
    batch_shape = x.shape
    combined = _build_combined(bucket_assignment, emb0, emb1, emb2, emb3, emb4,
                               W3.T, b3, W4.T, b4)
    x_flat = x.reshape(-1).astype(jnp.int32)
    out = _sc_gather(combined, x_flat)
    return out.reshape(*batch_shape, BASE)
